# baseline (speedup 1.0000x reference)
import dataclasses
import functools

import jax
import jax.numpy as jnp
from jax import lax
from jax.experimental import pallas as pl
from jax.experimental.pallas import tpu as pltpu
from jax.experimental.pallas import tpu_sc as plsc

N = 10000
F_IN = 128
H1 = 256
H2 = 64
F_OUT = 41
E = 320000
ETOT = E + N
NW = 32
NB = 43
BK = 240
EPAD = NW * NB * BK
NPAD = 10240
RPT = NPAD // 16
RB = 1024

_mesh = plsc.VectorSubcoreMesh(core_axis_name="c", subcore_axis_name="s")
_f32 = jnp.float32

_sc_params = pltpu.CompilerParams()
if "needs_layout_passes" in pltpu.CompilerParams.__dataclass_fields__:
    _sc_params = dataclasses.replace(_sc_params, needs_layout_passes=False)
if "use_tc_tiling_on_sc" in pltpu.CompilerParams.__dataclass_fields__:
    _sc_params = dataclasses.replace(_sc_params, use_tc_tiling_on_sc=False)


def _k1_body(x_ref, w_ref, asv_ref, adv_ref,
             h1_ref, h2_ref, h3_ref, h4_ref, as_ref, ad_ref):
    h = jnp.dot(x_ref[...], w_ref[...], preferred_element_type=_f32)
    h1_ref[...] = h[:, 0:64]
    h2_ref[...] = h[:, 64:128]
    h3_ref[...] = h[:, 128:192]
    h4_ref[...] = h[:, 192:256]
    as_ref[...] = jnp.sum(h * asv_ref[...], axis=1)
    ad_ref[...] = jnp.sum(h * adv_ref[...], axis=1)


def _k1(x_pad, W1, a1_src, a1_dst):
    grid = (NPAD // RB,)
    return pl.pallas_call(
        _k1_body,
        grid=grid,
        in_specs=[
            pl.BlockSpec((RB, F_IN), lambda i: (i, 0)),
            pl.BlockSpec((F_IN, H1), lambda i: (0, 0)),
            pl.BlockSpec((1, H1), lambda i: (0, 0)),
            pl.BlockSpec((1, H1), lambda i: (0, 0)),
        ],
        out_specs=[pl.BlockSpec((RB, 64), lambda i: (i, 0))] * 4 + [
            pl.BlockSpec((RB,), lambda i: (i,)),
            pl.BlockSpec((RB,), lambda i: (i,)),
        ],
        out_shape=[jax.ShapeDtypeStruct((NPAD, 64), _f32)] * 4 + [
            jax.ShapeDtypeStruct((NPAD,), _f32),
            jax.ShapeDtypeStruct((NPAD,), _f32),
        ],
    )(x_pad, W1, a1_src[None, :], a1_dst[None, :])


def _k2_body(p1_ref, p2_ref, p3_ref, p4_ref, dp_ref, b1_ref, w2_ref,
             a2s_ref, a2d_ref, h2_ref, as2_ref, ad2_ref):
    inv_den = (1.0 / jnp.sum(dp_ref[...], axis=0))[:, None]
    h2 = jnp.zeros((RB, H2), _f32)
    for q, p_ref in enumerate((p1_ref, p2_ref, p3_ref, p4_ref)):
        xq = jnp.maximum(
            (p_ref[0] + p_ref[1]) * inv_den + b1_ref[0, q * 64:(q + 1) * 64],
            0.0)
        h2 = h2 + jnp.dot(xq, w2_ref[q * 64:(q + 1) * 64, :],
                          preferred_element_type=_f32)
    h2_ref[...] = h2
    as2_ref[...] = jnp.sum(h2 * a2s_ref[...], axis=1)
    ad2_ref[...] = jnp.sum(h2 * a2d_ref[...], axis=1)


def _k2(ps, dp, b1, W2, a2_src, a2_dst):
    grid = (NPAD // RB,)
    return pl.pallas_call(
        _k2_body,
        grid=grid,
        in_specs=[pl.BlockSpec((2, RB, 64), lambda i: (0, i, 0))] * 4 + [
            pl.BlockSpec((NW, RB), lambda i: (0, i)),
            pl.BlockSpec((1, H1), lambda i: (0, 0)),
            pl.BlockSpec((H1, H2), lambda i: (0, 0)),
            pl.BlockSpec((1, H2), lambda i: (0, 0)),
            pl.BlockSpec((1, H2), lambda i: (0, 0)),
        ],
        out_specs=[
            pl.BlockSpec((RB, H2), lambda i: (i, 0)),
            pl.BlockSpec((RB,), lambda i: (i,)),
            pl.BlockSpec((RB,), lambda i: (i,)),
        ],
        out_shape=[
            jax.ShapeDtypeStruct((NPAD, H2), _f32),
            jax.ShapeDtypeStruct((NPAD,), _f32),
            jax.ShapeDtypeStruct((NPAD,), _f32),
        ],
    )(*ps, dp, b1[None, :], W2, a2_src[None, :], a2_dst[None, :])


def _k3_body(o2p_ref, dp_ref, b2_ref, fcw_ref, fcb_ref, y_ref, emb_ref):
    inv_den = (1.0 / jnp.sum(dp_ref[...], axis=0))[:, None]
    o2 = (o2p_ref[0] + o2p_ref[1]) * inv_den + b2_ref[...]
    emb = jnp.maximum(o2, 0.0)
    emb_ref[...] = emb
    y_ref[...] = (jnp.dot(emb, fcw_ref[...], preferred_element_type=_f32)
                  + fcb_ref[...])


def _k3(o2p, dp, b2, fcW, fcb):
    rb = RB
    grid = (NPAD // rb,)
    return pl.pallas_call(
        _k3_body,
        grid=grid,
        in_specs=[
            pl.BlockSpec((2, rb, H2), lambda i: (0, i, 0)),
            pl.BlockSpec((NW, rb), lambda i: (0, i)),
            pl.BlockSpec((1, H2), lambda i: (0, 0)),
            pl.BlockSpec((H2, F_OUT), lambda i: (0, 0)),
            pl.BlockSpec((1, F_OUT), lambda i: (0, 0)),
        ],
        out_specs=[
            pl.BlockSpec((rb, F_OUT), lambda i: (i, 0)),
            pl.BlockSpec((rb, H2), lambda i: (i, 0)),
        ],
        out_shape=[
            jax.ShapeDtypeStruct((NPAD, F_OUT), _f32),
            jax.ShapeDtypeStruct((NPAD, H2), _f32),
        ],
    )(o2p, dp, b2[None, :], fcW, fcb[None, :])


def _edge_weight(asv, adv, sv, dv):
    a1 = plsc.load_gather(asv, [sv])
    a2 = plsc.load_gather(adv, [dv])
    e = a1 + a2
    e = jnp.where(e > 0.0, e, 0.2 * e)
    return jnp.exp(e)


def _make_kb(Fh, with_den):
    FB = Fh // 16

    out_type = [jax.ShapeDtypeStruct((2, NPAD, Fh), _f32)]
    scratch = [
        pltpu.VMEM((NPAD,), _f32),
        pltpu.VMEM((NPAD,), _f32),
        pltpu.VMEM((NB, BK), jnp.int32),
        pltpu.VMEM((NB, BK), jnp.int32),
        pltpu.VMEM((BK, Fh), _f32),
        pltpu.VMEM((BK, Fh), _f32),
        pltpu.VMEM_SHARED((NPAD, Fh), _f32),
        pltpu.SemaphoreType.DMA,
        pltpu.SemaphoreType.DMA,
        pltpu.SemaphoreType.DMA,
        pltpu.SemaphoreType.DMA,
        pltpu.SemaphoreType.DMA,
        pltpu.SemaphoreType.DMA,
    ]
    if with_den:
        out_type.append(jax.ShapeDtypeStruct((NW, NPAD), _f32))
        scratch.append(pltpu.VMEM((NPAD,), _f32))
    else:
        scratch.insert(6, pltpu.VMEM((BK, Fh), _f32))

    @functools.partial(
        pl.kernel,
        out_type=tuple(out_type) if with_den else out_type[0],
        mesh=_mesh,
        scratch_types=scratch,
        compiler_params=_sc_params,
    )
    def kb(as_hbm, ad_hbm, src_hbm, dst_hbm, h_hbm, out_hbm, *rest):
        if with_den:
            (den_hbm, asv, adv, srcv, dstv, rows0, rows1, osp,
             g0, g1, g2, s0, s1, s2, den) = rest
        else:
            (asv, adv, srcv, dstv, rows0, rows1, rows2, osp,
             g0, g1, g2, s0, s1, s2) = rest
        cid = lax.axis_index("c")
        sid = lax.axis_index("s")
        wid = sid * 2 + cid
        pltpu.sync_copy(as_hbm, asv)
        pltpu.sync_copy(ad_hbm, adv)
        pltpu.sync_copy(src_hbm.at[wid], srcv)
        pltpu.sync_copy(dst_hbm.at[wid], dstv)

        if with_den:
            @pl.loop(0, NPAD, step=16)
            def _(i):
                den[pl.ds(i, 16)] = jnp.zeros((16,), _f32)

        @pl.loop(0, BK)
        def _(j):
            for f in range(FB):
                rows0[j, pl.ds(f * 16, 16)] = jnp.zeros((16,), _f32)

        @pl.loop(0, RPT, step=160)
        def _(r):
            pltpu.sync_copy(rows0.at[pl.ds(0, 160)],
                            osp.at[pl.ds(sid * RPT + r, 160)])

        plsc.subcore_barrier()

        def scale(b, rows):
            @pl.loop(0, BK, step=16)
            def _(k):
                dv = dstv[b, pl.ds(k, 16)]
                w16 = _edge_weight(asv, adv, srcv[b, pl.ds(k, 16)], dv)
                if with_den:
                    plsc.addupdate_scatter(den, [dv], w16)
                for l in range(16):
                    s = w16[l]
                    for f in range(FB):
                        rows[k + l, pl.ds(f * 16, 16)] = (
                            rows[k + l, pl.ds(f * 16, 16)] * s)

        if with_den:
            pltpu.async_copy(h_hbm.at[srcv.at[0]], rows0, g0)

            @pl.loop(0, NB - 1, step=2)
            def _(b):
                pltpu.async_copy(h_hbm.at[srcv.at[b + 1]], rows1, g1)
                pltpu.make_async_copy(h_hbm.at[srcv.at[b]], rows0, g0).wait()
                scale(b, rows0)
                pltpu.sync_copy(rows0, osp.at[dstv.at[b]], add=True)
                pltpu.async_copy(h_hbm.at[srcv.at[b + 2]], rows0, g0)
                pltpu.make_async_copy(
                    h_hbm.at[srcv.at[b + 1]], rows1, g1).wait()
                scale(b + 1, rows1)
                pltpu.sync_copy(rows1, osp.at[dstv.at[b + 1]], add=True)

            pltpu.make_async_copy(h_hbm.at[srcv.at[NB - 1]], rows0, g0).wait()
            scale(NB - 1, rows0)
            pltpu.sync_copy(rows0, osp.at[dstv.at[NB - 1]], add=True)
            pltpu.sync_copy(den, den_hbm.at[wid])
            plsc.subcore_barrier()
            pltpu.sync_copy(osp.at[pl.ds(sid * RPT, RPT)],
                            out_hbm.at[cid, pl.ds(sid * RPT, RPT)])
            return

        pltpu.async_copy(h_hbm.at[srcv.at[0]], rows0, g0)

        @pl.loop(0, NB // 3)
        def _(i):
            b3 = i * 3

            @pl.when(i >= 1)
            def _():
                pltpu.make_async_copy(rows1, osp.at[dstv.at[0]], s1).wait()
            pltpu.async_copy(h_hbm.at[srcv.at[b3 + 1]], rows1, g1)
            pltpu.make_async_copy(h_hbm.at[srcv.at[b3]], rows0, g0).wait()
            scale(b3, rows0)
            pltpu.async_copy(rows0, osp.at[dstv.at[b3]], s0, add=True)

            @pl.when(i >= 1)
            def _():
                pltpu.make_async_copy(rows2, osp.at[dstv.at[0]], s2).wait()
            pltpu.async_copy(h_hbm.at[srcv.at[b3 + 2]], rows2, g2)
            pltpu.make_async_copy(h_hbm.at[srcv.at[b3 + 1]], rows1, g1).wait()
            scale(b3 + 1, rows1)
            pltpu.async_copy(rows1, osp.at[dstv.at[b3 + 1]], s1, add=True)

            pltpu.make_async_copy(rows0, osp.at[dstv.at[0]], s0).wait()
            pltpu.async_copy(h_hbm.at[srcv.at[b3 + 3]], rows0, g0)
            pltpu.make_async_copy(h_hbm.at[srcv.at[b3 + 2]], rows2, g2).wait()
            scale(b3 + 2, rows2)
            pltpu.async_copy(rows2, osp.at[dstv.at[b3 + 2]], s2, add=True)

        pltpu.make_async_copy(h_hbm.at[srcv.at[NB - 1]], rows0, g0).wait()
        scale(NB - 1, rows0)
        pltpu.sync_copy(rows0, osp.at[dstv.at[NB - 1]], add=True)
        pltpu.make_async_copy(rows1, osp.at[dstv.at[0]], s1).wait()
        pltpu.make_async_copy(rows2, osp.at[dstv.at[0]], s2).wait()

        plsc.subcore_barrier()
        pltpu.sync_copy(osp.at[pl.ds(sid * RPT, RPT)],
                        out_hbm.at[cid, pl.ds(sid * RPT, RPT)])

    return kb


_kbd = _make_kb(64, True)
_kb = _make_kb(64, False)


def kernel(x, edge_index, W1, a1_src, a1_dst, b1, W2, a2_src, a2_dst, b2,
           fcW, fcb):
    loop = jnp.arange(N, dtype=jnp.int32)
    src = jnp.concatenate(
        [edge_index[0], loop, jnp.zeros((EPAD - ETOT,), jnp.int32)])
    dst = jnp.concatenate(
        [edge_index[1], loop,
         N + jnp.arange(EPAD - ETOT, dtype=jnp.int32)])
    src3 = src.reshape(NW, NB, BK)
    dst3 = dst.reshape(NW, NB, BK)
    x_pad = jnp.pad(x, ((0, NPAD - N), (0, 0)))

    h1a, h1b, h1c, h1d, as1, ad1 = _k1(x_pad, W1, a1_src, a1_dst)
    pa, dp1 = _kbd(as1, ad1, src3, dst3, h1a)
    ps = [pa] + [_kb(as1, ad1, src3, dst3, hq) for hq in (h1b, h1c, h1d)]

    h2, as2, ad2 = _k2(ps, dp1, b1, W2, a2_src, a2_dst)
    o2p, dp2 = _kbd(as2, ad2, src3, dst3, h2)

    y_pad, emb_pad = _k3(o2p, dp2, b2, fcW, fcb)
    return (y_pad[:N], emb_pad[:N])


# speedup vs baseline: 1.0301x; 1.0301x over previous
import dataclasses
import functools

import jax
import jax.numpy as jnp
from jax import lax
from jax.experimental import pallas as pl
from jax.experimental.pallas import tpu as pltpu
from jax.experimental.pallas import tpu_sc as plsc

N = 10000
F_IN = 128
H1 = 256
H2 = 64
F_OUT = 41
E = 320000
ETOT = E + N
NW = 32
NB = 43
BK = 240
EPAD = NW * NB * BK
NPAD = 10240
RPT = NPAD // 16
RB = 1024

_mesh = plsc.VectorSubcoreMesh(core_axis_name="c", subcore_axis_name="s")
_f32 = jnp.float32

_sc_params = pltpu.CompilerParams()
if "needs_layout_passes" in pltpu.CompilerParams.__dataclass_fields__:
    _sc_params = dataclasses.replace(_sc_params, needs_layout_passes=False)
if "use_tc_tiling_on_sc" in pltpu.CompilerParams.__dataclass_fields__:
    _sc_params = dataclasses.replace(_sc_params, use_tc_tiling_on_sc=False)


def _k1_body(x_ref, w_ref, asv_ref, adv_ref,
             h1_ref, h2_ref, h3_ref, h4_ref, as_ref, ad_ref):
    h = jnp.dot(x_ref[...], w_ref[...], preferred_element_type=_f32)
    h1_ref[...] = h[:, 0:64]
    h2_ref[...] = h[:, 64:128]
    h3_ref[...] = h[:, 128:192]
    h4_ref[...] = h[:, 192:256]
    as_ref[...] = jnp.sum(h * asv_ref[...], axis=1)
    ad_ref[...] = jnp.sum(h * adv_ref[...], axis=1)


def _k1(x_pad, W1, a1_src, a1_dst):
    grid = (NPAD // RB,)
    return pl.pallas_call(
        _k1_body,
        grid=grid,
        in_specs=[
            pl.BlockSpec((RB, F_IN), lambda i: (i, 0)),
            pl.BlockSpec((F_IN, H1), lambda i: (0, 0)),
            pl.BlockSpec((1, H1), lambda i: (0, 0)),
            pl.BlockSpec((1, H1), lambda i: (0, 0)),
        ],
        out_specs=[pl.BlockSpec((RB, 64), lambda i: (i, 0))] * 4 + [
            pl.BlockSpec((RB,), lambda i: (i,)),
            pl.BlockSpec((RB,), lambda i: (i,)),
        ],
        out_shape=[jax.ShapeDtypeStruct((NPAD, 64), _f32)] * 4 + [
            jax.ShapeDtypeStruct((NPAD,), _f32),
            jax.ShapeDtypeStruct((NPAD,), _f32),
        ],
    )(x_pad, W1, a1_src[None, :], a1_dst[None, :])


def _k2_body(p1_ref, p2_ref, p3_ref, p4_ref, dp_ref, b1_ref, w2_ref,
             a2s_ref, a2d_ref, h2_ref, as2_ref, ad2_ref):
    inv_den = (1.0 / jnp.sum(dp_ref[...], axis=0))[:, None]
    h2 = jnp.zeros((RB, H2), _f32)
    for q, p_ref in enumerate((p1_ref, p2_ref, p3_ref, p4_ref)):
        xq = jnp.maximum(
            (p_ref[0] + p_ref[1]) * inv_den + b1_ref[0, q * 64:(q + 1) * 64],
            0.0)
        h2 = h2 + jnp.dot(xq, w2_ref[q * 64:(q + 1) * 64, :],
                          preferred_element_type=_f32)
    h2_ref[...] = h2
    as2_ref[...] = jnp.sum(h2 * a2s_ref[...], axis=1)
    ad2_ref[...] = jnp.sum(h2 * a2d_ref[...], axis=1)


def _k2(ps, dp, b1, W2, a2_src, a2_dst):
    grid = (NPAD // RB,)
    return pl.pallas_call(
        _k2_body,
        grid=grid,
        in_specs=[pl.BlockSpec((2, RB, 64), lambda i: (0, i, 0))] * 4 + [
            pl.BlockSpec((NW, RB), lambda i: (0, i)),
            pl.BlockSpec((1, H1), lambda i: (0, 0)),
            pl.BlockSpec((H1, H2), lambda i: (0, 0)),
            pl.BlockSpec((1, H2), lambda i: (0, 0)),
            pl.BlockSpec((1, H2), lambda i: (0, 0)),
        ],
        out_specs=[
            pl.BlockSpec((RB, H2), lambda i: (i, 0)),
            pl.BlockSpec((RB,), lambda i: (i,)),
            pl.BlockSpec((RB,), lambda i: (i,)),
        ],
        out_shape=[
            jax.ShapeDtypeStruct((NPAD, H2), _f32),
            jax.ShapeDtypeStruct((NPAD,), _f32),
            jax.ShapeDtypeStruct((NPAD,), _f32),
        ],
    )(*ps, dp, b1[None, :], W2, a2_src[None, :], a2_dst[None, :])


def _k3_body(o2p_ref, dp_ref, b2_ref, fcw_ref, fcb_ref, y_ref, emb_ref):
    inv_den = (1.0 / jnp.sum(dp_ref[...], axis=0))[:, None]
    o2 = (o2p_ref[0] + o2p_ref[1]) * inv_den + b2_ref[...]
    emb = jnp.maximum(o2, 0.0)
    emb_ref[...] = emb
    y_ref[...] = (jnp.dot(emb, fcw_ref[...], preferred_element_type=_f32)
                  + fcb_ref[...])


def _k3(o2p, dp, b2, fcW, fcb):
    rb = RB
    grid = (NPAD // rb,)
    return pl.pallas_call(
        _k3_body,
        grid=grid,
        in_specs=[
            pl.BlockSpec((2, rb, H2), lambda i: (0, i, 0)),
            pl.BlockSpec((NW, rb), lambda i: (0, i)),
            pl.BlockSpec((1, H2), lambda i: (0, 0)),
            pl.BlockSpec((H2, F_OUT), lambda i: (0, 0)),
            pl.BlockSpec((1, F_OUT), lambda i: (0, 0)),
        ],
        out_specs=[
            pl.BlockSpec((rb, F_OUT), lambda i: (i, 0)),
            pl.BlockSpec((rb, H2), lambda i: (i, 0)),
        ],
        out_shape=[
            jax.ShapeDtypeStruct((NPAD, F_OUT), _f32),
            jax.ShapeDtypeStruct((NPAD, H2), _f32),
        ],
    )(o2p, dp, b2[None, :], fcW, fcb[None, :])


def _edge_weight(asv, adv, sv, dv):
    a1 = plsc.load_gather(asv, [sv])
    a2 = plsc.load_gather(adv, [dv])
    e = a1 + a2
    e = jnp.where(e > 0.0, e, 0.2 * e)
    return jnp.exp(e)


def _make_kb(Fh, with_den):
    FB = Fh // 16

    nq = 1 if with_den else 3
    out_type = [jax.ShapeDtypeStruct((2, NPAD, Fh), _f32)] * nq
    scratch = [
        pltpu.VMEM((NPAD,), _f32),
        pltpu.VMEM((NPAD,), _f32),
        pltpu.VMEM((NB, BK), jnp.int32),
        pltpu.VMEM((NB, BK), jnp.int32),
        pltpu.VMEM((BK, Fh), _f32),
        pltpu.VMEM((BK, Fh), _f32),
        pltpu.VMEM_SHARED((NPAD, Fh), _f32),
        pltpu.SemaphoreType.DMA,
        pltpu.SemaphoreType.DMA,
        pltpu.SemaphoreType.DMA,
        pltpu.SemaphoreType.DMA,
        pltpu.SemaphoreType.DMA,
        pltpu.SemaphoreType.DMA,
    ]
    if with_den:
        out_type.append(jax.ShapeDtypeStruct((NW, NPAD), _f32))
        scratch.append(pltpu.VMEM((NPAD,), _f32))
    else:
        scratch.insert(6, pltpu.VMEM((BK, Fh), _f32))

    @functools.partial(
        pl.kernel,
        out_type=tuple(out_type),
        mesh=_mesh,
        scratch_types=scratch,
        compiler_params=_sc_params,
    )
    def kb(as_hbm, ad_hbm, src_hbm, dst_hbm, *h_out_rest):
        if with_den:
            (h_hbm, out_hbm, den_hbm, asv, adv, srcv, dstv, rows0, rows1,
             osp, g0, g1, g2, s0, s1, s2, den) = h_out_rest
            quarters = [(h_hbm, out_hbm)]
        else:
            (ha, hb, hc, oa, ob, oc, asv, adv, srcv, dstv,
             rows0, rows1, rows2, osp, g0, g1, g2, s0, s1, s2) = h_out_rest
            quarters = [(ha, oa), (hb, ob), (hc, oc)]
        cid = lax.axis_index("c")
        sid = lax.axis_index("s")
        wid = sid * 2 + cid
        pltpu.sync_copy(as_hbm, asv)
        pltpu.sync_copy(ad_hbm, adv)
        pltpu.sync_copy(src_hbm.at[wid], srcv)
        pltpu.sync_copy(dst_hbm.at[wid], dstv)

        if with_den:
            @pl.loop(0, NPAD, step=16)
            def _(i):
                den[pl.ds(i, 16)] = jnp.zeros((16,), _f32)

        def zero_osp():
            @pl.loop(0, BK)
            def _(j):
                for f in range(FB):
                    rows0[j, pl.ds(f * 16, 16)] = jnp.zeros((16,), _f32)

            @pl.loop(0, RPT, step=160)
            def _(r):
                pltpu.sync_copy(rows0.at[pl.ds(0, 160)],
                                osp.at[pl.ds(sid * RPT + r, 160)])

        def scale(b, rows):
            @pl.loop(0, BK, step=16)
            def _(k):
                dv = dstv[b, pl.ds(k, 16)]
                w16 = _edge_weight(asv, adv, srcv[b, pl.ds(k, 16)], dv)
                if with_den:
                    plsc.addupdate_scatter(den, [dv], w16)
                for l in range(16):
                    s = w16[l]
                    for f in range(FB):
                        rows[k + l, pl.ds(f * 16, 16)] = (
                            rows[k + l, pl.ds(f * 16, 16)] * s)

        def main_2buf(h_hbm):
            pltpu.async_copy(h_hbm.at[srcv.at[0]], rows0, g0)

            @pl.loop(0, NB - 1, step=2)
            def _(b):
                pltpu.async_copy(h_hbm.at[srcv.at[b + 1]], rows1, g1)
                pltpu.make_async_copy(h_hbm.at[srcv.at[b]], rows0, g0).wait()
                scale(b, rows0)
                pltpu.sync_copy(rows0, osp.at[dstv.at[b]], add=True)
                pltpu.async_copy(h_hbm.at[srcv.at[b + 2]], rows0, g0)
                pltpu.make_async_copy(
                    h_hbm.at[srcv.at[b + 1]], rows1, g1).wait()
                scale(b + 1, rows1)
                pltpu.sync_copy(rows1, osp.at[dstv.at[b + 1]], add=True)

            pltpu.make_async_copy(h_hbm.at[srcv.at[NB - 1]], rows0, g0).wait()
            scale(NB - 1, rows0)
            pltpu.sync_copy(rows0, osp.at[dstv.at[NB - 1]], add=True)

        def main_ring(h_hbm):
            pltpu.async_copy(h_hbm.at[srcv.at[0]], rows0, g0)

            @pl.loop(0, NB // 3)
            def _(i):
                b3 = i * 3

                @pl.when(i >= 1)
                def _():
                    pltpu.make_async_copy(
                        rows1, osp.at[dstv.at[0]], s1).wait()
                pltpu.async_copy(h_hbm.at[srcv.at[b3 + 1]], rows1, g1)
                pltpu.make_async_copy(h_hbm.at[srcv.at[b3]], rows0, g0).wait()
                scale(b3, rows0)
                pltpu.async_copy(rows0, osp.at[dstv.at[b3]], s0, add=True)

                @pl.when(i >= 1)
                def _():
                    pltpu.make_async_copy(
                        rows2, osp.at[dstv.at[0]], s2).wait()
                pltpu.async_copy(h_hbm.at[srcv.at[b3 + 2]], rows2, g2)
                pltpu.make_async_copy(
                    h_hbm.at[srcv.at[b3 + 1]], rows1, g1).wait()
                scale(b3 + 1, rows1)
                pltpu.async_copy(rows1, osp.at[dstv.at[b3 + 1]], s1, add=True)

                pltpu.make_async_copy(rows0, osp.at[dstv.at[0]], s0).wait()
                pltpu.async_copy(h_hbm.at[srcv.at[b3 + 3]], rows0, g0)
                pltpu.make_async_copy(
                    h_hbm.at[srcv.at[b3 + 2]], rows2, g2).wait()
                scale(b3 + 2, rows2)
                pltpu.async_copy(rows2, osp.at[dstv.at[b3 + 2]], s2, add=True)

            pltpu.make_async_copy(h_hbm.at[srcv.at[NB - 1]], rows0, g0).wait()
            scale(NB - 1, rows0)
            pltpu.sync_copy(rows0, osp.at[dstv.at[NB - 1]], add=True)
            pltpu.make_async_copy(rows1, osp.at[dstv.at[0]], s1).wait()
            pltpu.make_async_copy(rows2, osp.at[dstv.at[0]], s2).wait()

        for qi, (h_hbm, out_hbm) in enumerate(quarters):
            if qi > 0:
                plsc.subcore_barrier()
            zero_osp()
            plsc.subcore_barrier()
            if with_den:
                main_2buf(h_hbm)
                pltpu.sync_copy(den, den_hbm.at[wid])
            else:
                main_ring(h_hbm)
            plsc.subcore_barrier()
            pltpu.sync_copy(osp.at[pl.ds(sid * RPT, RPT)],
                            out_hbm.at[cid, pl.ds(sid * RPT, RPT)])

    return kb


_kbd = _make_kb(64, True)
_kb = _make_kb(64, False)


def kernel(x, edge_index, W1, a1_src, a1_dst, b1, W2, a2_src, a2_dst, b2,
           fcW, fcb):
    loop = jnp.arange(N, dtype=jnp.int32)
    src = jnp.concatenate(
        [edge_index[0], loop, jnp.zeros((EPAD - ETOT,), jnp.int32)])
    dst = jnp.concatenate(
        [edge_index[1], loop,
         N + jnp.arange(EPAD - ETOT, dtype=jnp.int32)])
    src3 = src.reshape(NW, NB, BK)
    dst3 = dst.reshape(NW, NB, BK)
    x_pad = jnp.pad(x, ((0, NPAD - N), (0, 0)))

    h1a, h1b, h1c, h1d, as1, ad1 = _k1(x_pad, W1, a1_src, a1_dst)
    pa, dp1 = _kbd(as1, ad1, src3, dst3, h1a)
    pb, pc, pd = _kb(as1, ad1, src3, dst3, h1b, h1c, h1d)
    ps = [pa, pb, pc, pd]

    h2, as2, ad2 = _k2(ps, dp1, b1, W2, a2_src, a2_dst)
    o2p, dp2 = _kbd(as2, ad2, src3, dst3, h2)

    y_pad, emb_pad = _k3(o2p, dp2, b2, fcW, fcb)
    return (y_pad[:N], emb_pad[:N])


# speedup vs baseline: 1.0978x; 1.0658x over previous
import dataclasses
import functools

import jax
import jax.numpy as jnp
from jax import lax
from jax.experimental import pallas as pl
from jax.experimental.pallas import tpu as pltpu
from jax.experimental.pallas import tpu_sc as plsc

N = 10000
F_IN = 128
H1 = 256
H2 = 64
F_OUT = 41
E = 320000
ETOT = E + N
NW = 32
NB = 43
BK = 240
EPAD = NW * NB * BK
NPAD = 10240
RPT = NPAD // 16
RB = 1024

_mesh = plsc.VectorSubcoreMesh(core_axis_name="c", subcore_axis_name="s")
_f32 = jnp.float32

_sc_params = pltpu.CompilerParams()
if "needs_layout_passes" in pltpu.CompilerParams.__dataclass_fields__:
    _sc_params = dataclasses.replace(_sc_params, needs_layout_passes=False)
if "use_tc_tiling_on_sc" in pltpu.CompilerParams.__dataclass_fields__:
    _sc_params = dataclasses.replace(_sc_params, use_tc_tiling_on_sc=False)


def _k1_body(x_ref, w_ref, asv_ref, adv_ref,
             h1_ref, h2_ref, h3_ref, h4_ref, as_ref, ad_ref):
    h = jnp.dot(x_ref[...], w_ref[...], preferred_element_type=_f32)
    h1_ref[...] = h[:, 0:64]
    h2_ref[...] = h[:, 64:128]
    h3_ref[...] = h[:, 128:192]
    h4_ref[...] = h[:, 192:256]
    as_ref[...] = jnp.sum(h * asv_ref[...], axis=1)
    ad_ref[...] = jnp.sum(h * adv_ref[...], axis=1)


def _k1(x_pad, W1, a1_src, a1_dst):
    grid = (NPAD // RB,)
    return pl.pallas_call(
        _k1_body,
        grid=grid,
        in_specs=[
            pl.BlockSpec((RB, F_IN), lambda i: (i, 0)),
            pl.BlockSpec((F_IN, H1), lambda i: (0, 0)),
            pl.BlockSpec((1, H1), lambda i: (0, 0)),
            pl.BlockSpec((1, H1), lambda i: (0, 0)),
        ],
        out_specs=[pl.BlockSpec((RB, 64), lambda i: (i, 0))] * 4 + [
            pl.BlockSpec((RB,), lambda i: (i,)),
            pl.BlockSpec((RB,), lambda i: (i,)),
        ],
        out_shape=[jax.ShapeDtypeStruct((NPAD, 64), _f32)] * 4 + [
            jax.ShapeDtypeStruct((NPAD,), _f32),
            jax.ShapeDtypeStruct((NPAD,), _f32),
        ],
    )(x_pad, W1, a1_src[None, :], a1_dst[None, :])


def _k2_body(p1_ref, p2_ref, p3_ref, p4_ref, dp_ref, b1_ref, w2_ref,
             a2s_ref, a2d_ref, h2_ref, as2_ref, ad2_ref):
    inv_den = (1.0 / jnp.sum(dp_ref[...], axis=0))[:, None]
    h2 = jnp.zeros((RB, H2), _f32)
    for q, p_ref in enumerate((p1_ref, p2_ref, p3_ref, p4_ref)):
        xq = jnp.maximum(
            (p_ref[0] + p_ref[1]) * inv_den + b1_ref[0, q * 64:(q + 1) * 64],
            0.0)
        h2 = h2 + jnp.dot(xq, w2_ref[q * 64:(q + 1) * 64, :],
                          preferred_element_type=_f32)
    h2_ref[...] = h2
    as2_ref[...] = jnp.sum(h2 * a2s_ref[...], axis=1)
    ad2_ref[...] = jnp.sum(h2 * a2d_ref[...], axis=1)


def _k2(ps, dp, b1, W2, a2_src, a2_dst):
    grid = (NPAD // RB,)
    return pl.pallas_call(
        _k2_body,
        grid=grid,
        in_specs=[pl.BlockSpec((2, RB, 64), lambda i: (0, i, 0))] * 4 + [
            pl.BlockSpec((NW, RB), lambda i: (0, i)),
            pl.BlockSpec((1, H1), lambda i: (0, 0)),
            pl.BlockSpec((H1, H2), lambda i: (0, 0)),
            pl.BlockSpec((1, H2), lambda i: (0, 0)),
            pl.BlockSpec((1, H2), lambda i: (0, 0)),
        ],
        out_specs=[
            pl.BlockSpec((RB, H2), lambda i: (i, 0)),
            pl.BlockSpec((RB,), lambda i: (i,)),
            pl.BlockSpec((RB,), lambda i: (i,)),
        ],
        out_shape=[
            jax.ShapeDtypeStruct((NPAD, H2), _f32),
            jax.ShapeDtypeStruct((NPAD,), _f32),
            jax.ShapeDtypeStruct((NPAD,), _f32),
        ],
    )(*ps, dp, b1[None, :], W2, a2_src[None, :], a2_dst[None, :])


def _k3_body(o2p_ref, dp_ref, b2_ref, fcw_ref, fcb_ref, y_ref, emb_ref):
    inv_den = (1.0 / jnp.sum(dp_ref[...], axis=0))[:, None]
    o2 = (o2p_ref[0] + o2p_ref[1]) * inv_den + b2_ref[...]
    emb = jnp.maximum(o2, 0.0)
    emb_ref[...] = emb
    y_ref[...] = (jnp.dot(emb, fcw_ref[...], preferred_element_type=_f32)
                  + fcb_ref[...])


def _k3(o2p, dp, b2, fcW, fcb):
    rb = RB
    grid = (NPAD // rb,)
    return pl.pallas_call(
        _k3_body,
        grid=grid,
        in_specs=[
            pl.BlockSpec((2, rb, H2), lambda i: (0, i, 0)),
            pl.BlockSpec((NW, rb), lambda i: (0, i)),
            pl.BlockSpec((1, H2), lambda i: (0, 0)),
            pl.BlockSpec((H2, F_OUT), lambda i: (0, 0)),
            pl.BlockSpec((1, F_OUT), lambda i: (0, 0)),
        ],
        out_specs=[
            pl.BlockSpec((rb, F_OUT), lambda i: (i, 0)),
            pl.BlockSpec((rb, H2), lambda i: (i, 0)),
        ],
        out_shape=[
            jax.ShapeDtypeStruct((NPAD, F_OUT), _f32),
            jax.ShapeDtypeStruct((NPAD, H2), _f32),
        ],
    )(o2p, dp, b2[None, :], fcW, fcb[None, :])


def _edge_weight(asv, adv, sv, dv):
    a1 = plsc.load_gather(asv, [sv])
    a2 = plsc.load_gather(adv, [dv])
    e = a1 + a2
    e = jnp.where(e > 0.0, e, 0.2 * e)
    return jnp.exp(e)


def _make_kb(Fh, with_den):
    FB = Fh // 16

    nq = 1 if with_den else 3
    out_type = [jax.ShapeDtypeStruct((2, NPAD, Fh), _f32)] * nq
    scratch = [
        pltpu.VMEM((NPAD,), _f32),
        pltpu.VMEM((NPAD,), _f32),
        pltpu.VMEM((NB, BK), jnp.int32),
        pltpu.VMEM((NB, BK), jnp.int32),
        pltpu.VMEM((BK, Fh), _f32),
        pltpu.VMEM((BK, Fh), _f32),
        pltpu.VMEM_SHARED((NPAD, Fh), _f32),
        pltpu.SemaphoreType.DMA,
        pltpu.SemaphoreType.DMA,
        pltpu.SemaphoreType.DMA,
        pltpu.SemaphoreType.DMA,
        pltpu.SemaphoreType.DMA,
        pltpu.SemaphoreType.DMA,
    ]
    if with_den:
        out_type.append(jax.ShapeDtypeStruct((NW, NPAD), _f32))
        scratch.append(pltpu.VMEM((NPAD,), _f32))
    else:
        scratch.insert(6, pltpu.VMEM((BK, Fh), _f32))

    @functools.partial(
        pl.kernel,
        out_type=tuple(out_type),
        mesh=_mesh,
        scratch_types=scratch,
        compiler_params=_sc_params,
    )
    def kb(as_hbm, ad_hbm, src_hbm, dst_hbm, *h_out_rest):
        if with_den:
            (h_hbm, out_hbm, den_hbm, asv, adv, srcv, dstv, rows0, rows1,
             osp, g0, g1, g2, s0, s1, s2, den) = h_out_rest
            quarters = [(h_hbm, out_hbm)]
        else:
            (ha, hb, hc, oa, ob, oc, asv, adv, srcv, dstv,
             rows0, rows1, rows2, osp, g0, g1, g2, s0, s1, s2) = h_out_rest
            quarters = [(ha, oa), (hb, ob), (hc, oc)]
        cid = lax.axis_index("c")
        sid = lax.axis_index("s")
        wid = sid * 2 + cid
        pltpu.sync_copy(as_hbm, asv)
        pltpu.sync_copy(ad_hbm, adv)
        pltpu.sync_copy(src_hbm.at[wid], srcv)
        pltpu.sync_copy(dst_hbm.at[wid], dstv)

        if with_den:
            @pl.loop(0, NPAD, step=16)
            def _(i):
                den[pl.ds(i, 16)] = jnp.zeros((16,), _f32)

        def zero_osp():
            @pl.loop(0, BK)
            def _(j):
                for f in range(FB):
                    rows0[j, pl.ds(f * 16, 16)] = jnp.zeros((16,), _f32)

            @pl.loop(0, RPT, step=160)
            def _(r):
                pltpu.sync_copy(rows0.at[pl.ds(0, 160)],
                                osp.at[pl.ds(sid * RPT + r, 160)])

        def scale_body(b, rows, k):
            dv = dstv[b, pl.ds(k, 16)]
            w16 = _edge_weight(asv, adv, srcv[b, pl.ds(k, 16)], dv)
            if with_den:
                plsc.addupdate_scatter(den, [dv], w16)
            for l in range(16):
                s = w16[l]
                for f in range(FB):
                    rows[k + l, pl.ds(f * 16, 16)] = (
                        rows[k + l, pl.ds(f * 16, 16)] * s)

        def scale(b, rows):
            if with_den:
                @pl.loop(0, BK, step=16)
                def _(k):
                    scale_body(b, rows, k)
            else:
                @plsc.parallel_loop(0, BK, step=16, unroll=2)
                def _(k):
                    scale_body(b, rows, k)

        def main_2buf(h_hbm):
            pltpu.async_copy(h_hbm.at[srcv.at[0]], rows0, g0)

            @pl.loop(0, NB - 1, step=2)
            def _(b):
                pltpu.async_copy(h_hbm.at[srcv.at[b + 1]], rows1, g1)
                pltpu.make_async_copy(h_hbm.at[srcv.at[b]], rows0, g0).wait()
                scale(b, rows0)
                pltpu.sync_copy(rows0, osp.at[dstv.at[b]], add=True)
                pltpu.async_copy(h_hbm.at[srcv.at[b + 2]], rows0, g0)
                pltpu.make_async_copy(
                    h_hbm.at[srcv.at[b + 1]], rows1, g1).wait()
                scale(b + 1, rows1)
                pltpu.sync_copy(rows1, osp.at[dstv.at[b + 1]], add=True)

            pltpu.make_async_copy(h_hbm.at[srcv.at[NB - 1]], rows0, g0).wait()
            scale(NB - 1, rows0)
            pltpu.sync_copy(rows0, osp.at[dstv.at[NB - 1]], add=True)

        def main_ring(h_hbm):
            pltpu.async_copy(h_hbm.at[srcv.at[0]], rows0, g0)

            @pl.loop(0, NB // 3)
            def _(i):
                b3 = i * 3

                @pl.when(i >= 1)
                def _():
                    pltpu.make_async_copy(
                        rows1, osp.at[dstv.at[0]], s1).wait()
                pltpu.async_copy(h_hbm.at[srcv.at[b3 + 1]], rows1, g1)
                pltpu.make_async_copy(h_hbm.at[srcv.at[b3]], rows0, g0).wait()
                scale(b3, rows0)
                pltpu.async_copy(rows0, osp.at[dstv.at[b3]], s0, add=True)

                @pl.when(i >= 1)
                def _():
                    pltpu.make_async_copy(
                        rows2, osp.at[dstv.at[0]], s2).wait()
                pltpu.async_copy(h_hbm.at[srcv.at[b3 + 2]], rows2, g2)
                pltpu.make_async_copy(
                    h_hbm.at[srcv.at[b3 + 1]], rows1, g1).wait()
                scale(b3 + 1, rows1)
                pltpu.async_copy(rows1, osp.at[dstv.at[b3 + 1]], s1, add=True)

                pltpu.make_async_copy(rows0, osp.at[dstv.at[0]], s0).wait()
                pltpu.async_copy(h_hbm.at[srcv.at[b3 + 3]], rows0, g0)
                pltpu.make_async_copy(
                    h_hbm.at[srcv.at[b3 + 2]], rows2, g2).wait()
                scale(b3 + 2, rows2)
                pltpu.async_copy(rows2, osp.at[dstv.at[b3 + 2]], s2, add=True)

            pltpu.make_async_copy(h_hbm.at[srcv.at[NB - 1]], rows0, g0).wait()
            scale(NB - 1, rows0)
            pltpu.sync_copy(rows0, osp.at[dstv.at[NB - 1]], add=True)
            pltpu.make_async_copy(rows1, osp.at[dstv.at[0]], s1).wait()
            pltpu.make_async_copy(rows2, osp.at[dstv.at[0]], s2).wait()

        for qi, (h_hbm, out_hbm) in enumerate(quarters):
            if qi > 0:
                plsc.subcore_barrier()
            zero_osp()
            plsc.subcore_barrier()
            if with_den:
                main_2buf(h_hbm)
                pltpu.sync_copy(den, den_hbm.at[wid])
            else:
                main_ring(h_hbm)
            plsc.subcore_barrier()
            pltpu.sync_copy(osp.at[pl.ds(sid * RPT, RPT)],
                            out_hbm.at[cid, pl.ds(sid * RPT, RPT)])

    return kb


_kbd = _make_kb(64, True)
_kb = _make_kb(64, False)


def kernel(x, edge_index, W1, a1_src, a1_dst, b1, W2, a2_src, a2_dst, b2,
           fcW, fcb):
    loop = jnp.arange(N, dtype=jnp.int32)
    src = jnp.concatenate(
        [edge_index[0], loop, jnp.zeros((EPAD - ETOT,), jnp.int32)])
    dst = jnp.concatenate(
        [edge_index[1], loop,
         N + jnp.arange(EPAD - ETOT, dtype=jnp.int32)])
    src3 = src.reshape(NW, NB, BK)
    dst3 = dst.reshape(NW, NB, BK)
    x_pad = jnp.pad(x, ((0, NPAD - N), (0, 0)))

    h1a, h1b, h1c, h1d, as1, ad1 = _k1(x_pad, W1, a1_src, a1_dst)
    pa, dp1 = _kbd(as1, ad1, src3, dst3, h1a)
    pb, pc, pd = _kb(as1, ad1, src3, dst3, h1b, h1c, h1d)
    ps = [pa, pb, pc, pd]

    h2, as2, ad2 = _k2(ps, dp1, b1, W2, a2_src, a2_dst)
    o2p, dp2 = _kbd(as2, ad2, src3, dst3, h2)

    y_pad, emb_pad = _k3(o2p, dp2, b2, fcW, fcb)
    return (y_pad[:N], emb_pad[:N])


# speedup vs baseline: 1.2133x; 1.1052x over previous
import dataclasses
import functools

import jax
import jax.numpy as jnp
from jax import lax
from jax.experimental import pallas as pl
from jax.experimental.pallas import tpu as pltpu
from jax.experimental.pallas import tpu_sc as plsc

N = 10000
F_IN = 128
H1 = 256
H2 = 64
F_OUT = 41
E = 320000
ETOT = E + N
NW = 32
NB = 43
BK = 240
EPAD = NW * NB * BK
NPAD = 10240
RPT = NPAD // 16
RB = 1024

_mesh = plsc.VectorSubcoreMesh(core_axis_name="c", subcore_axis_name="s")
_f32 = jnp.float32

_sc_params = pltpu.CompilerParams()
if "needs_layout_passes" in pltpu.CompilerParams.__dataclass_fields__:
    _sc_params = dataclasses.replace(_sc_params, needs_layout_passes=False)
if "use_tc_tiling_on_sc" in pltpu.CompilerParams.__dataclass_fields__:
    _sc_params = dataclasses.replace(_sc_params, use_tc_tiling_on_sc=False)


def _k1_body(x_ref, w_ref, asv_ref, adv_ref,
             h1_ref, h2_ref, h3_ref, h4_ref, as_ref, ad_ref):
    h = jnp.dot(x_ref[...], w_ref[...], preferred_element_type=_f32)
    h1_ref[...] = h[:, 0:64]
    h2_ref[...] = h[:, 64:128]
    h3_ref[...] = h[:, 128:192]
    h4_ref[...] = h[:, 192:256]
    as_ref[...] = jnp.sum(h * asv_ref[...], axis=1)
    ad_ref[...] = jnp.sum(h * adv_ref[...], axis=1)


def _k1(x_pad, W1, a1_src, a1_dst):
    grid = (NPAD // RB,)
    return pl.pallas_call(
        _k1_body,
        grid=grid,
        in_specs=[
            pl.BlockSpec((RB, F_IN), lambda i: (i, 0)),
            pl.BlockSpec((F_IN, H1), lambda i: (0, 0)),
            pl.BlockSpec((1, H1), lambda i: (0, 0)),
            pl.BlockSpec((1, H1), lambda i: (0, 0)),
        ],
        out_specs=[pl.BlockSpec((RB, 64), lambda i: (i, 0))] * 4 + [
            pl.BlockSpec((RB,), lambda i: (i,)),
            pl.BlockSpec((RB,), lambda i: (i,)),
        ],
        out_shape=[jax.ShapeDtypeStruct((NPAD, 64), _f32)] * 4 + [
            jax.ShapeDtypeStruct((NPAD,), _f32),
            jax.ShapeDtypeStruct((NPAD,), _f32),
        ],
    )(x_pad, W1, a1_src[None, :], a1_dst[None, :])


def _k2_body(p1_ref, p2_ref, p3_ref, p4_ref, dp_ref, b1_ref, w2_ref,
             a2s_ref, a2d_ref, h2_ref, as2_ref, ad2_ref):
    inv_den = (1.0 / jnp.sum(dp_ref[...], axis=0))[:, None]
    h2 = jnp.zeros((RB, H2), _f32)
    for q, p_ref in enumerate((p1_ref, p2_ref, p3_ref, p4_ref)):
        xq = jnp.maximum(
            (p_ref[0] + p_ref[1]) * inv_den + b1_ref[0, q * 64:(q + 1) * 64],
            0.0)
        h2 = h2 + jnp.dot(xq, w2_ref[q * 64:(q + 1) * 64, :],
                          preferred_element_type=_f32)
    h2_ref[...] = h2
    as2_ref[...] = jnp.sum(h2 * a2s_ref[...], axis=1)
    ad2_ref[...] = jnp.sum(h2 * a2d_ref[...], axis=1)


def _k2(ps, dp, b1, W2, a2_src, a2_dst):
    grid = (NPAD // RB,)
    return pl.pallas_call(
        _k2_body,
        grid=grid,
        in_specs=[pl.BlockSpec((2, RB, 64), lambda i: (0, i, 0))] * 4 + [
            pl.BlockSpec((NW, RB), lambda i: (0, i)),
            pl.BlockSpec((1, H1), lambda i: (0, 0)),
            pl.BlockSpec((H1, H2), lambda i: (0, 0)),
            pl.BlockSpec((1, H2), lambda i: (0, 0)),
            pl.BlockSpec((1, H2), lambda i: (0, 0)),
        ],
        out_specs=[
            pl.BlockSpec((RB, H2), lambda i: (i, 0)),
            pl.BlockSpec((RB,), lambda i: (i,)),
            pl.BlockSpec((RB,), lambda i: (i,)),
        ],
        out_shape=[
            jax.ShapeDtypeStruct((NPAD, H2), _f32),
            jax.ShapeDtypeStruct((NPAD,), _f32),
            jax.ShapeDtypeStruct((NPAD,), _f32),
        ],
    )(*ps, dp, b1[None, :], W2, a2_src[None, :], a2_dst[None, :])


def _k3_body(o2p_ref, dp_ref, b2_ref, fcw_ref, fcb_ref, y_ref, emb_ref):
    inv_den = (1.0 / jnp.sum(dp_ref[...], axis=0))[:, None]
    o2 = (o2p_ref[0] + o2p_ref[1]) * inv_den + b2_ref[...]
    emb = jnp.maximum(o2, 0.0)
    emb_ref[...] = emb
    y_ref[...] = (jnp.dot(emb, fcw_ref[...], preferred_element_type=_f32)
                  + fcb_ref[...])


def _k3(o2p, dp, b2, fcW, fcb):
    rb = RB
    grid = (NPAD // rb,)
    return pl.pallas_call(
        _k3_body,
        grid=grid,
        in_specs=[
            pl.BlockSpec((2, rb, H2), lambda i: (0, i, 0)),
            pl.BlockSpec((NW, rb), lambda i: (0, i)),
            pl.BlockSpec((1, H2), lambda i: (0, 0)),
            pl.BlockSpec((H2, F_OUT), lambda i: (0, 0)),
            pl.BlockSpec((1, F_OUT), lambda i: (0, 0)),
        ],
        out_specs=[
            pl.BlockSpec((rb, F_OUT), lambda i: (i, 0)),
            pl.BlockSpec((rb, H2), lambda i: (i, 0)),
        ],
        out_shape=[
            jax.ShapeDtypeStruct((NPAD, F_OUT), _f32),
            jax.ShapeDtypeStruct((NPAD, H2), _f32),
        ],
    )(o2p, dp, b2[None, :], fcW, fcb[None, :])


def _edge_weight(asv, adv, sv, dv):
    a1 = plsc.load_gather(asv, [sv])
    a2 = plsc.load_gather(adv, [dv])
    e = a1 + a2
    e = jnp.where(e > 0.0, e, 0.2 * e)
    return jnp.exp(e)


def _make_kb(Fh, with_den):
    FB = Fh // 16

    nq = 1 if with_den else 3
    out_type = [jax.ShapeDtypeStruct((2, NPAD, Fh), _f32)] * nq
    scratch = [
        pltpu.VMEM((NPAD,), _f32),
        pltpu.VMEM((NPAD,), _f32),
        pltpu.VMEM((NB, BK), jnp.int32),
        pltpu.VMEM((NB, BK), jnp.int32),
        pltpu.VMEM((BK, Fh), _f32),
        pltpu.VMEM((BK, Fh), _f32),
        pltpu.VMEM_SHARED((NPAD, Fh), _f32),
        pltpu.SemaphoreType.DMA,
        pltpu.SemaphoreType.DMA,
        pltpu.SemaphoreType.DMA,
        pltpu.SemaphoreType.DMA,
        pltpu.SemaphoreType.DMA,
        pltpu.SemaphoreType.DMA,
    ]
    if with_den:
        out_type.append(jax.ShapeDtypeStruct((NW, NPAD), _f32))
        scratch.append(pltpu.VMEM((NPAD,), _f32))
    else:
        scratch.insert(6, pltpu.VMEM((BK, Fh), _f32))

    @functools.partial(
        pl.kernel,
        out_type=tuple(out_type),
        mesh=_mesh,
        scratch_types=scratch,
        compiler_params=_sc_params,
    )
    def kb(as_hbm, ad_hbm, src_hbm, dst_hbm, *h_out_rest):
        if with_den:
            (h_hbm, out_hbm, den_hbm, asv, adv, srcv, dstv, rows0, rows1,
             osp, g0, g1, g2, s0, s1, s2, den) = h_out_rest
            quarters = [(h_hbm, out_hbm)]
        else:
            (ha, hb, hc, oa, ob, oc, asv, adv, srcv, dstv,
             rows0, rows1, rows2, osp, g0, g1, g2, s0, s1, s2) = h_out_rest
            quarters = [(ha, oa), (hb, ob), (hc, oc)]
        cid = lax.axis_index("c")
        sid = lax.axis_index("s")
        wid = sid * 2 + cid
        pltpu.sync_copy(as_hbm, asv)
        pltpu.sync_copy(ad_hbm, adv)
        pltpu.sync_copy(src_hbm.at[wid], srcv)
        pltpu.sync_copy(dst_hbm.at[wid], dstv)

        if with_den:
            @pl.loop(0, NPAD, step=16)
            def _(i):
                den[pl.ds(i, 16)] = jnp.zeros((16,), _f32)

        def zero_osp():
            @pl.loop(0, BK)
            def _(j):
                for f in range(FB):
                    rows0[j, pl.ds(f * 16, 16)] = jnp.zeros((16,), _f32)

            @pl.loop(0, RPT, step=160)
            def _(r):
                pltpu.sync_copy(rows0.at[pl.ds(0, 160)],
                                osp.at[pl.ds(sid * RPT + r, 160)])

        def scale_body(b, rows, k):
            dv = dstv[b, pl.ds(k, 16)]
            w16 = _edge_weight(asv, adv, srcv[b, pl.ds(k, 16)], dv)
            if with_den:
                plsc.addupdate_scatter(den, [dv], w16)
            for l in range(16):
                s = w16[l]
                for f in range(FB):
                    rows[k + l, pl.ds(f * 16, 16)] = (
                        rows[k + l, pl.ds(f * 16, 16)] * s)

        def scale(b, rows):
            @plsc.parallel_loop(0, BK, step=16, unroll=3)
            def _(k):
                scale_body(b, rows, k)

        def main_2buf(h_hbm):
            pltpu.async_copy(h_hbm.at[srcv.at[0]], rows0, g0)

            @pl.loop(0, NB - 1, step=2)
            def _(b):
                pltpu.async_copy(h_hbm.at[srcv.at[b + 1]], rows1, g1)
                pltpu.make_async_copy(h_hbm.at[srcv.at[b]], rows0, g0).wait()
                scale(b, rows0)
                pltpu.sync_copy(rows0, osp.at[dstv.at[b]], add=True)
                pltpu.async_copy(h_hbm.at[srcv.at[b + 2]], rows0, g0)
                pltpu.make_async_copy(
                    h_hbm.at[srcv.at[b + 1]], rows1, g1).wait()
                scale(b + 1, rows1)
                pltpu.sync_copy(rows1, osp.at[dstv.at[b + 1]], add=True)

            pltpu.make_async_copy(h_hbm.at[srcv.at[NB - 1]], rows0, g0).wait()
            scale(NB - 1, rows0)
            pltpu.sync_copy(rows0, osp.at[dstv.at[NB - 1]], add=True)

        def main_ring(h_hbm):
            pltpu.async_copy(h_hbm.at[srcv.at[0]], rows0, g0)

            @pl.loop(0, NB // 3)
            def _(i):
                b3 = i * 3

                @pl.when(i >= 1)
                def _():
                    pltpu.make_async_copy(
                        rows1, osp.at[dstv.at[0]], s1).wait()
                pltpu.async_copy(h_hbm.at[srcv.at[b3 + 1]], rows1, g1)
                pltpu.make_async_copy(h_hbm.at[srcv.at[b3]], rows0, g0).wait()
                scale(b3, rows0)
                pltpu.async_copy(rows0, osp.at[dstv.at[b3]], s0, add=True)

                @pl.when(i >= 1)
                def _():
                    pltpu.make_async_copy(
                        rows2, osp.at[dstv.at[0]], s2).wait()
                pltpu.async_copy(h_hbm.at[srcv.at[b3 + 2]], rows2, g2)
                pltpu.make_async_copy(
                    h_hbm.at[srcv.at[b3 + 1]], rows1, g1).wait()
                scale(b3 + 1, rows1)
                pltpu.async_copy(rows1, osp.at[dstv.at[b3 + 1]], s1, add=True)

                pltpu.make_async_copy(rows0, osp.at[dstv.at[0]], s0).wait()
                pltpu.async_copy(h_hbm.at[srcv.at[b3 + 3]], rows0, g0)
                pltpu.make_async_copy(
                    h_hbm.at[srcv.at[b3 + 2]], rows2, g2).wait()
                scale(b3 + 2, rows2)
                pltpu.async_copy(rows2, osp.at[dstv.at[b3 + 2]], s2, add=True)

            pltpu.make_async_copy(h_hbm.at[srcv.at[NB - 1]], rows0, g0).wait()
            scale(NB - 1, rows0)
            pltpu.sync_copy(rows0, osp.at[dstv.at[NB - 1]], add=True)
            pltpu.make_async_copy(rows1, osp.at[dstv.at[0]], s1).wait()
            pltpu.make_async_copy(rows2, osp.at[dstv.at[0]], s2).wait()

        for qi, (h_hbm, out_hbm) in enumerate(quarters):
            if qi > 0:
                plsc.subcore_barrier()
            zero_osp()
            plsc.subcore_barrier()
            if with_den:
                main_2buf(h_hbm)
                pltpu.sync_copy(den, den_hbm.at[wid])
            else:
                main_ring(h_hbm)
            plsc.subcore_barrier()
            pltpu.sync_copy(osp.at[pl.ds(sid * RPT, RPT)],
                            out_hbm.at[cid, pl.ds(sid * RPT, RPT)])

    return kb


_kbd = _make_kb(64, True)
_kb = _make_kb(64, False)


def kernel(x, edge_index, W1, a1_src, a1_dst, b1, W2, a2_src, a2_dst, b2,
           fcW, fcb):
    loop = jnp.arange(N, dtype=jnp.int32)
    src = jnp.concatenate(
        [edge_index[0], loop, jnp.zeros((EPAD - ETOT,), jnp.int32)])
    dst = jnp.concatenate(
        [edge_index[1], loop,
         N + jnp.arange(EPAD - ETOT, dtype=jnp.int32)])
    src3 = src.reshape(NW, NB, BK)
    dst3 = dst.reshape(NW, NB, BK)
    x_pad = jnp.pad(x, ((0, NPAD - N), (0, 0)))

    h1a, h1b, h1c, h1d, as1, ad1 = _k1(x_pad, W1, a1_src, a1_dst)
    pa, dp1 = _kbd(as1, ad1, src3, dst3, h1a)
    pb, pc, pd = _kb(as1, ad1, src3, dst3, h1b, h1c, h1d)
    ps = [pa, pb, pc, pd]

    h2, as2, ad2 = _k2(ps, dp1, b1, W2, a2_src, a2_dst)
    o2p, dp2 = _kbd(as2, ad2, src3, dst3, h2)

    y_pad, emb_pad = _k3(o2p, dp2, b2, fcW, fcb)
    return (y_pad[:N], emb_pad[:N])
